# baseline (speedup 1.0000x reference)
import numpy as np
import jax
import jax.numpy as jnp
from jax import lax
from jax.experimental import pallas as pl
from jax.experimental.pallas import tpu as pltpu

C1 = 96
CE = 96
CAND = C1 * (C1 - 1) // 2
LN = 128
ROWS = (CAND + LN - 1) // LN
NPAD = ROWS * LN
BIG = np.int32(2 ** 30)
OUT_BLK = 24


def _build_enc_table():
    enc = np.full(NPAD, 1e9, np.float32)
    k = 0
    for i in range(C1):
        for j in range(i + 1, C1):
            enc[k] = i * C1 + j
            k += 1
    return enc.reshape(ROWS, LN)


_ENC_NP = _build_enc_table()


def _body(logits_ref, gumb_ref, enc_ref, x_ref, w_ref, bias_ref, out_ref,
          ii_ref, jj_ref):
    b = pl.program_id(0)
    g = pl.program_id(1)
    inv_n = np.float32(1.0 / (x_ref.shape[2] * x_ref.shape[3]))

    @pl.when((b == 0) & (g == 0))
    def _select():
        sc = logits_ref[...] + gumb_ref[...]
        kbits = lax.bitcast_convert_type(sc, jnp.int32)
        sortable = jnp.where(kbits < 0, kbits ^ jnp.int32(0x7FFFFFFF), kbits)
        enc = enc_ref[...]

        def bs(_, lohi):
            lo, hi = lohi
            mid = (lo >> 1) + (hi >> 1) + (lo & hi & 1)
            cnt = jnp.sum(jnp.where(sortable > mid, 1, 0))
            take = cnt >= CE
            return (jnp.where(take, mid, lo), jnp.where(take, hi, mid))

        lo, hi = lax.fori_loop(
            0, 32, bs, (jnp.int32(-(2 ** 31)), jnp.int32(2 ** 31 - 1)))
        t = hi

        gt = sortable > t
        ntie = CE - jnp.sum(jnp.where(gt, 1, 0))
        BIGF = jnp.float32(1e9)

        def vmin2(a):
            m = jnp.min(a, axis=0, keepdims=True)
            return jnp.min(m, axis=1, keepdims=True)

        def tie(_, kvtk):
            kv, tk = kvtk
            key = vmin2(tk)
            return (jnp.where(enc == key, enc, kv),
                    jnp.where(tk == key, BIGF, tk))

        kv0, _ = lax.fori_loop(
            0, ntie, tie,
            (jnp.where(gt, enc, BIGF), jnp.where(sortable == t, enc, BIGF)))

        def extract(e, kv):
            key = vmin2(kv)
            key_s = jnp.min(kv).astype(jnp.int32)
            ii_ref[e] = key_s // C1
            jj_ref[e] = key_s % C1
            return jnp.where(kv == key, BIGF, kv)

        lax.fori_loop(0, CE, extract, kv0, unroll=8)

    def finish(c_loc, c_glob, s1, s2, plane):
        mean = s1 * inv_n
        var = s2 * inv_n - mean * mean
        scale = w_ref[c_glob] * lax.rsqrt(var + np.float32(1e-5))
        out_ref[0, c_loc] = plane * scale + (bias_ref[c_glob] - mean * scale)

    def norm_store(c_loc, plane, c_glob):
        s1 = jnp.sum(plane)
        s2 = jnp.sum(plane * plane)
        finish(c_loc, c_glob, s1, s2, plane)

    @pl.when(g * OUT_BLK < C1)
    def _copy():
        def cc(l, carry):
            c = g * OUT_BLK + l
            norm_store(l, x_ref[0, c], c)
            return carry

        lax.fori_loop(0, OUT_BLK, cc, 0, unroll=8)

    @pl.when(g * OUT_BLK >= C1)
    def _prod():
        def ee(l, carry):
            e = g * OUT_BLK + l - C1
            i = ii_ref[e]
            j = jj_ref[e]
            p = x_ref[0, i] * x_ref[0, j]
            out_ref[0, l] = p
            s1 = jnp.sum(p)
            s2 = jnp.sum(p * p)
            finish(l, C1 + e, s1, s2, out_ref[0, l])
            return carry

        lax.fori_loop(0, OUT_BLK, ee, 0, unroll=8)


def kernel(x, logits, tau, in_weight, in_bias):
    del tau
    B, C, H, W = x.shape

    gumb = -jnp.log(jax.random.exponential(jax.random.key(42), logits.shape,
                                           dtype=jnp.float32))
    pad = NPAD - CAND
    logits_p = jnp.concatenate(
        [logits.astype(jnp.float32), jnp.zeros((pad,), jnp.float32)]
    ).reshape(ROWS, LN)
    gumb_p = jnp.concatenate(
        [gumb, jnp.full((pad,), -1e30, jnp.float32)]
    ).reshape(ROWS, LN)
    enc = jnp.asarray(_ENC_NP)

    out = pl.pallas_call(
        _body,
        grid=(B, (C1 + CE) // OUT_BLK),
        in_specs=[
            pl.BlockSpec((ROWS, LN), lambda b, g: (0, 0)),
            pl.BlockSpec((ROWS, LN), lambda b, g: (0, 0)),
            pl.BlockSpec((ROWS, LN), lambda b, g: (0, 0)),
            pl.BlockSpec((1, C, H, W), lambda b, g: (b, 0, 0, 0)),
            pl.BlockSpec(memory_space=pltpu.SMEM),
            pl.BlockSpec(memory_space=pltpu.SMEM),
        ],
        out_specs=pl.BlockSpec((1, OUT_BLK, H, W),
                               lambda b, g: (b, g, 0, 0)),
        out_shape=jax.ShapeDtypeStruct((B, C1 + CE, H, W), jnp.float32),
        scratch_shapes=[
            pltpu.SMEM((CE,), jnp.int32),
            pltpu.SMEM((CE,), jnp.int32),
        ],
        compiler_params=pltpu.CompilerParams(
            dimension_semantics=("arbitrary", "arbitrary")),
    )(logits_p, gumb_p, enc, x, in_weight.astype(jnp.float32),
      in_bias.astype(jnp.float32))
    return out


# speedup vs baseline: 1.0653x; 1.0653x over previous
import numpy as np
import jax
import jax.numpy as jnp
from jax import lax
from jax.experimental import pallas as pl
from jax.experimental.pallas import tpu as pltpu

C1 = 96
CE = 96
CAND = C1 * (C1 - 1) // 2
LN = 128
ROWS = (CAND + LN - 1) // LN
NPAD = ROWS * LN
BIG = np.int32(2 ** 30)
OUT_BLK = 24


def _build_enc_table():
    enc = np.full(NPAD, 1e9, np.float32)
    k = 0
    for i in range(C1):
        for j in range(i + 1, C1):
            enc[k] = i * C1 + j
            k += 1
    return enc.reshape(ROWS, LN)


_ENC_NP = _build_enc_table()


def _body(logits_ref, gumb_ref, enc_ref, x_ref, w_ref, bias_ref, out_ref,
          ii_ref, jj_ref):
    b = pl.program_id(0)
    g = pl.program_id(1)
    inv_n = np.float32(1.0 / (x_ref.shape[2] * x_ref.shape[3]))

    @pl.when((b == 0) & (g == 0))
    def _select():
        sc = logits_ref[...] + gumb_ref[...]
        kbits = lax.bitcast_convert_type(sc, jnp.int32)
        sortable = jnp.where(kbits < 0, kbits ^ jnp.int32(0x7FFFFFFF), kbits)
        enc = enc_ref[...]

        def bs(_, lohi):
            lo, hi = lohi
            mid = (lo >> 1) + (hi >> 1) + (lo & hi & 1)
            cnt = jnp.sum(jnp.where(sortable > mid, 1, 0))
            take = cnt >= CE
            return (jnp.where(take, mid, lo), jnp.where(take, hi, mid))

        lo, hi = lax.fori_loop(
            0, 32, bs, (jnp.int32(-(2 ** 31)), jnp.int32(2 ** 31 - 1)))
        t = hi

        gt = sortable > t
        ntie = CE - jnp.sum(jnp.where(gt, 1, 0))
        BIGF = jnp.float32(1e9)

        def vmin2(a):
            m = jnp.min(a, axis=0, keepdims=True)
            return jnp.min(m, axis=1, keepdims=True)

        def tie(_, kvtk):
            kv, tk = kvtk
            key = vmin2(tk)
            return (jnp.where(enc == key, enc, kv),
                    jnp.where(tk == key, BIGF, tk))

        kv0, _ = lax.fori_loop(
            0, ntie, tie,
            (jnp.where(gt, enc, BIGF), jnp.where(sortable == t, enc, BIGF)))

        def extract(e, kv):
            key = vmin2(kv)
            key_s = jnp.min(kv).astype(jnp.int32)
            ii_ref[e] = key_s // C1
            jj_ref[e] = key_s % C1
            return jnp.where(kv == key, BIGF, kv)

        lax.fori_loop(0, CE, extract, kv0, unroll=8)

    def norm_store(c_loc, plane, c_glob):
        s1 = jnp.sum(plane)
        s2 = jnp.sum(plane * plane)
        mean = s1 * inv_n
        var = s2 * inv_n - mean * mean
        scale = w_ref[c_glob] * lax.rsqrt(var + np.float32(1e-5))
        out_ref[0, c_loc] = plane * scale + (bias_ref[c_glob] - mean * scale)

    @pl.when(g * OUT_BLK < C1)
    def _copy():
        def cc(l, carry):
            c = g * OUT_BLK + l
            norm_store(l, x_ref[0, c], c)
            return carry

        lax.fori_loop(0, OUT_BLK, cc, 0, unroll=12)

    @pl.when(g * OUT_BLK >= C1)
    def _prod():
        def ee(l, carry):
            e = g * OUT_BLK + l - C1
            i = ii_ref[e]
            j = jj_ref[e]
            norm_store(l, x_ref[0, i] * x_ref[0, j], C1 + e)
            return carry

        lax.fori_loop(0, OUT_BLK, ee, 0, unroll=12)


def kernel(x, logits, tau, in_weight, in_bias):
    del tau
    B, C, H, W = x.shape

    gumb = -jnp.log(jax.random.exponential(jax.random.key(42), logits.shape,
                                           dtype=jnp.float32))
    pad = NPAD - CAND
    logits_p = jnp.concatenate(
        [logits.astype(jnp.float32), jnp.zeros((pad,), jnp.float32)]
    ).reshape(ROWS, LN)
    gumb_p = jnp.concatenate(
        [gumb, jnp.full((pad,), -1e30, jnp.float32)]
    ).reshape(ROWS, LN)
    enc = jnp.asarray(_ENC_NP)

    out = pl.pallas_call(
        _body,
        grid=(B, (C1 + CE) // OUT_BLK),
        in_specs=[
            pl.BlockSpec((ROWS, LN), lambda b, g: (0, 0)),
            pl.BlockSpec((ROWS, LN), lambda b, g: (0, 0)),
            pl.BlockSpec((ROWS, LN), lambda b, g: (0, 0)),
            pl.BlockSpec((1, C, H, W), lambda b, g: (b, 0, 0, 0)),
            pl.BlockSpec(memory_space=pltpu.SMEM),
            pl.BlockSpec(memory_space=pltpu.SMEM),
        ],
        out_specs=pl.BlockSpec((1, OUT_BLK, H, W),
                               lambda b, g: (b, g, 0, 0)),
        out_shape=jax.ShapeDtypeStruct((B, C1 + CE, H, W), jnp.float32),
        scratch_shapes=[
            pltpu.SMEM((CE,), jnp.int32),
            pltpu.SMEM((CE,), jnp.int32),
        ],
        compiler_params=pltpu.CompilerParams(
            dimension_semantics=("arbitrary", "arbitrary")),
    )(logits_p, gumb_p, enc, x, in_weight.astype(jnp.float32),
      in_bias.astype(jnp.float32))
    return out


# speedup vs baseline: 1.0925x; 1.0255x over previous
import numpy as np
import jax
import jax.numpy as jnp
from jax import lax
from jax.experimental import pallas as pl
from jax.experimental.pallas import tpu as pltpu

C1 = 96
CE = 96
CAND = C1 * (C1 - 1) // 2
LN = 128
ROWS = (CAND + LN - 1) // LN
NPAD = ROWS * LN
BIG = np.int32(2 ** 30)
OUT_BLK = 24


def _build_enc_table():
    enc = np.full(NPAD, 1e9, np.float32)
    k = 0
    for i in range(C1):
        for j in range(i + 1, C1):
            enc[k] = i * C1 + j
            k += 1
    return enc.reshape(ROWS, LN)


_ENC_NP = _build_enc_table()


def _body(logits_ref, gumb_ref, enc_ref, x_ref, w_ref, bias_ref, out_ref,
          ii_ref, jj_ref):
    b = pl.program_id(0)
    g = pl.program_id(1)
    inv_n = np.float32(1.0 / (x_ref.shape[2] * x_ref.shape[3]))

    @pl.when((b == 0) & (g == 0))
    def _select():
        sc = logits_ref[...] + gumb_ref[...]
        kbits = lax.bitcast_convert_type(sc, jnp.int32)
        sortable = jnp.where(kbits < 0, kbits ^ jnp.int32(0x7FFFFFFF), kbits)
        enc = enc_ref[...]

        def bs(_, lohi):
            lo, hi = lohi
            mid = (lo >> 1) + (hi >> 1) + (lo & hi & 1)
            cnt = jnp.sum(jnp.where(sortable > mid, 1, 0))
            take = cnt >= CE
            return (jnp.where(take, mid, lo), jnp.where(take, hi, mid))

        lo, hi = lax.fori_loop(
            0, 32, bs, (jnp.int32(-(2 ** 31)), jnp.int32(2 ** 31 - 1)))
        t = hi

        gt = sortable > t
        ntie = CE - jnp.sum(jnp.where(gt, 1, 0))
        BIGF = jnp.float32(1e9)

        def vmin2(a):
            m = jnp.min(a, axis=0, keepdims=True)
            return jnp.min(m, axis=1, keepdims=True)

        def tie(_, kvtk):
            kv, tk = kvtk
            key = vmin2(tk)
            return (jnp.where(enc == key, enc, kv),
                    jnp.where(tk == key, BIGF, tk))

        kv0, _ = lax.fori_loop(
            0, ntie, tie,
            (jnp.where(gt, enc, BIGF), jnp.where(sortable == t, enc, BIGF)))

        def extract(e, kv):
            key = vmin2(kv)
            key_s = jnp.min(kv).astype(jnp.int32)
            ii_ref[e] = key_s // C1
            jj_ref[e] = key_s % C1
            return jnp.where(kv == key, BIGF, kv)

        lax.fori_loop(0, CE, extract, kv0, unroll=8)

    def norm_store(c_loc, plane, c_glob):
        s1 = jnp.sum(plane)
        s2 = jnp.sum(plane * plane)
        mean = s1 * inv_n
        var = s2 * inv_n - mean * mean
        scale = w_ref[c_glob] * lax.rsqrt(var + np.float32(1e-5))
        out_ref[0, c_loc] = plane * scale + (bias_ref[c_glob] - mean * scale)

    @pl.when(g * OUT_BLK < C1)
    def _copy():
        def cc(l, carry):
            c = g * OUT_BLK + l
            norm_store(l, x_ref[0, c], c)
            return carry

        lax.fori_loop(0, OUT_BLK, cc, 0, unroll=24)

    @pl.when(g * OUT_BLK >= C1)
    def _prod():
        def ee(l, carry):
            e = g * OUT_BLK + l - C1
            i = ii_ref[e]
            j = jj_ref[e]
            norm_store(l, x_ref[0, i] * x_ref[0, j], C1 + e)
            return carry

        lax.fori_loop(0, OUT_BLK, ee, 0, unroll=24)


def kernel(x, logits, tau, in_weight, in_bias):
    del tau
    B, C, H, W = x.shape

    gumb = -jnp.log(jax.random.exponential(jax.random.key(42), logits.shape,
                                           dtype=jnp.float32))
    pad = NPAD - CAND
    logits_p = jnp.concatenate(
        [logits.astype(jnp.float32), jnp.zeros((pad,), jnp.float32)]
    ).reshape(ROWS, LN)
    gumb_p = jnp.concatenate(
        [gumb, jnp.full((pad,), -1e30, jnp.float32)]
    ).reshape(ROWS, LN)
    enc = jnp.asarray(_ENC_NP)

    out = pl.pallas_call(
        _body,
        grid=(B, (C1 + CE) // OUT_BLK),
        in_specs=[
            pl.BlockSpec((ROWS, LN), lambda b, g: (0, 0)),
            pl.BlockSpec((ROWS, LN), lambda b, g: (0, 0)),
            pl.BlockSpec((ROWS, LN), lambda b, g: (0, 0)),
            pl.BlockSpec((1, C, H, W), lambda b, g: (b, 0, 0, 0)),
            pl.BlockSpec(memory_space=pltpu.SMEM),
            pl.BlockSpec(memory_space=pltpu.SMEM),
        ],
        out_specs=pl.BlockSpec((1, OUT_BLK, H, W),
                               lambda b, g: (b, g, 0, 0)),
        out_shape=jax.ShapeDtypeStruct((B, C1 + CE, H, W), jnp.float32),
        scratch_shapes=[
            pltpu.SMEM((CE,), jnp.int32),
            pltpu.SMEM((CE,), jnp.int32),
        ],
        compiler_params=pltpu.CompilerParams(
            dimension_semantics=("arbitrary", "arbitrary")),
    )(logits_p, gumb_p, enc, x, in_weight.astype(jnp.float32),
      in_bias.astype(jnp.float32))
    return out
